# baseline (speedup 1.0000x reference)
import jax
import jax.numpy as jnp
from jax.experimental import pallas as pl
from jax.experimental.pallas import tpu as pltpu

_INPUT_LEN = 200
_EMBED_DIM = 64
_BATCH = 4096
_ROWS = _BATCH * 100
_TR = 25600
_NB = _ROWS // _TR


def _bcast_body(pos_ref, out_ref, tile_ref, sem):
    tile_ref[...] = jnp.broadcast_to(
        pos_ref[...].reshape(100, 128)[None], (_TR // 100, 100, 128)
    ).reshape(_TR, 128)
    copies = [
        pltpu.make_async_copy(tile_ref, out_ref.at[pl.ds(j * _TR, _TR), :], sem)
        for j in range(_NB)
    ]
    for c in copies:
        c.start()
    for c in copies:
        c.wait()


def kernel(x, pos_table):
    del x
    pos_flat = pos_table.reshape(1, 12800)
    out = pl.pallas_call(
        _bcast_body,
        in_specs=[pl.BlockSpec((1, 12800), lambda: (0, 0))],
        out_specs=pl.BlockSpec(memory_space=pl.ANY),
        out_shape=jax.ShapeDtypeStruct((_ROWS, 128), jnp.float32),
        scratch_shapes=[
            pltpu.VMEM((_TR, 128), jnp.float32),
            pltpu.SemaphoreType.DMA,
        ],
    )(pos_flat)
    return out.reshape(_BATCH, _INPUT_LEN, _EMBED_DIM)


# speedup vs baseline: 1.3917x; 1.3917x over previous
import jax
import jax.numpy as jnp
from jax.experimental import pallas as pl
from jax.experimental.pallas import tpu as pltpu

_INPUT_LEN = 200
_EMBED_DIM = 64
_BATCH = 4096
_QR = 8
_NQ = _INPUT_LEN // _QR
_TB = 256
_NB = _BATCH // _TB


def _bcast_body(pos_ref, out_ref, tile_ref, sem):
    tile_ref[...] = jnp.broadcast_to(
        pos_ref[...].reshape(_NQ, 1, _QR, _EMBED_DIM),
        (_NQ, _TB, _QR, _EMBED_DIM),
    )
    copies = [
        pltpu.make_async_copy(
            tile_ref.at[j],
            out_ref.at[pl.ds(a * _TB, _TB), pl.ds(j * _QR, _QR), :],
            sem,
        )
        for j in range(_NQ)
        for a in range(_NB)
    ]
    for c in copies:
        c.start()
    for c in copies:
        c.wait()


def kernel(x, pos_table):
    del x
    return pl.pallas_call(
        _bcast_body,
        in_specs=[pl.BlockSpec((_INPUT_LEN, _EMBED_DIM), lambda: (0, 0))],
        out_specs=pl.BlockSpec(memory_space=pl.ANY),
        out_shape=jax.ShapeDtypeStruct((_BATCH, _INPUT_LEN, _EMBED_DIM), jnp.float32),
        scratch_shapes=[
            pltpu.VMEM((_NQ, _TB, _QR, _EMBED_DIM), jnp.float32),
            pltpu.SemaphoreType.DMA,
        ],
    )(pos_table)


# speedup vs baseline: 1.3956x; 1.0028x over previous
import jax
import jax.numpy as jnp
from jax.experimental import pallas as pl
from jax.experimental.pallas import tpu as pltpu

_INPUT_LEN = 200
_EMBED_DIM = 64
_BATCH = 4096
_TR = 256
_NB = _BATCH // _TR


def _bcast_body(pos_ref, out_ref, tile_ref, sem):
    tile_ref[...] = jnp.broadcast_to(pos_ref[...][None], tile_ref.shape)
    copies = [
        pltpu.make_async_copy(tile_ref, out_ref.at[pl.ds(j * _TR, _TR)], sem)
        for j in range(_NB)
    ]
    for c in copies:
        c.start()
    for c in copies:
        c.wait()


def kernel(x, pos_table):
    del x
    return pl.pallas_call(
        _bcast_body,
        in_specs=[pl.BlockSpec((_INPUT_LEN, _EMBED_DIM), lambda: (0, 0))],
        out_specs=pl.BlockSpec(memory_space=pl.ANY),
        compiler_params=pltpu.CompilerParams(needs_layout_passes=False),
        out_shape=jax.ShapeDtypeStruct((_BATCH, _INPUT_LEN, _EMBED_DIM), jnp.float32),
        scratch_shapes=[
            pltpu.VMEM((_TR, _INPUT_LEN, _EMBED_DIM), jnp.float32),
            pltpu.SemaphoreType.DMA,
        ],
    )(pos_table)


# speedup vs baseline: 1.9845x; 1.4220x over previous
import jax
import jax.numpy as jnp
from jax import lax
from jax.experimental import pallas as pl
from jax.experimental.pallas import tpu as pltpu, tpu_sc as plsc

_INPUT_LEN = 200
_EMBED_DIM = 64
_BATCH = 4096
_FLAT = _INPUT_LEN * _EMBED_DIM

_NC = 2
_NS = 16
_NW = _NC * _NS
_ROWS_PER_W = _BATCH // _NW
_REP = 8
_BLOCKS_PER_W = _ROWS_PER_W // _REP


def _make_sc_kernel():
    mesh = plsc.VectorSubcoreMesh(core_axis_name="c", subcore_axis_name="s")

    @pl.kernel(
        mesh=mesh,
        out_type=jax.ShapeDtypeStruct((_BATCH, _FLAT), jnp.float32),
        scratch_types=[
            pltpu.VMEM((_REP, _FLAT), jnp.float32),
            pltpu.SemaphoreType.DMA,
        ],
    )
    def sc_kernel(pos_hbm, out_hbm, tile_v, sem):
        wid = lax.axis_index("s") * _NC + lax.axis_index("c")
        base = wid * _ROWS_PER_W
        fills = [pltpu.async_copy(pos_hbm, tile_v.at[r], sem) for r in range(_REP)]
        for f in fills:
            f.wait()
        outs = [
            pltpu.async_copy(
                tile_v, out_hbm.at[pl.ds(base + j * _REP, _REP), :], sem
            )
            for j in range(_BLOCKS_PER_W)
        ]
        for c in outs:
            c.wait()

    return sc_kernel


_SC_KERNEL = _make_sc_kernel()


def kernel(x, pos_table):
    del x
    pos_flat = pos_table.reshape(_FLAT)
    out = _SC_KERNEL(pos_flat)
    return out.reshape(_BATCH, _INPUT_LEN, _EMBED_DIM)
